# baseline (speedup 1.0000x reference)
import numpy as np
import jax
import jax.numpy as jnp
from jax import lax
from jax.experimental import pallas as pl
from jax.experimental.pallas import tpu as pltpu


def _build_selectors(C: int, R: int) -> np.ndarray:
    CR, CC = C * R, C * C
    m = np.arange(CC)
    i, j = m // C, m % C
    k = np.arange(CR)
    w = np.zeros((2, CR, CC), np.float32)
    w[0, :, :] = (k[:, None] // R == i[None, :])
    w[1, :, :] = (k[:, None] // R == j[None, :])
    return w


def _product_body(xt_ref, w_ref, o_ref):
    bB = xt_ref.shape[0]
    C, R, F = xt_ref.shape[1:]
    F2 = F // 2
    CR = C * R
    RF2 = R * F2
    dn = (((0,), (0,)), ((), ()))
    ik = lax.broadcasted_iota(jnp.int32, (CR, RF2), 0) % R
    im = lax.broadcasted_iota(jnp.int32, (CR, RF2), 1) % R
    msk = (ik == im).astype(jnp.float32)
    H = RF2 // 2
    col = lax.broadcasted_iota(jnp.int32, (CR, H), 1) // R
    idx_e = col * 2
    idx_o = idx_e + 1

    def deint(x64, idx):
        lo = jnp.take_along_axis(x64[:, : F // 2], idx, axis=1)
        hi = jnp.take_along_axis(x64[:, F // 2 :], idx, axis=1)
        return jnp.concatenate([lo, hi], axis=1)

    for b in range(bB):
        x64 = xt_ref[b].reshape(CR, F)
        xe4 = deint(x64, idx_e) * msk
        xo4 = deint(x64, idx_o) * msk
        p = lax.dot_general(xe4, w_ref[0], dn,
                            preferred_element_type=jnp.float32)
        q = lax.dot_general(xo4, w_ref[1], dn,
                            preferred_element_type=jnp.float32)
        o_ref[b] = (p + q).reshape(F2, R, C * C)


def kernel(log_likelihoods):
    B, F, C, R = log_likelihoods.shape
    F2 = F // 2
    CR, CC = C * R, C * C
    xt = jnp.transpose(log_likelihoods, (0, 2, 3, 1))
    w = jnp.asarray(_build_selectors(C, R))
    bB = 32
    yt = pl.pallas_call(
        _product_body,
        grid=(B // bB,),
        in_specs=[
            pl.BlockSpec((bB, C, R, F), lambda b: (b, 0, 0, 0)),
            pl.BlockSpec((2, CR, CC), lambda b: (0, 0, 0)),
        ],
        out_specs=pl.BlockSpec((bB, F2, R, CC), lambda b: (b, 0, 0, 0)),
        out_shape=jax.ShapeDtypeStruct((B, F2, R, CC), jnp.float32),
        compiler_params=pltpu.CompilerParams(
            dimension_semantics=("arbitrary",),
        ),
    )(xt, w)
    return jnp.transpose(yt, (0, 1, 3, 2))


# speedup vs baseline: 1.0959x; 1.0959x over previous
import numpy as np
import jax
import jax.numpy as jnp
from jax import lax
from jax.experimental import pallas as pl
from jax.experimental.pallas import tpu as pltpu


def _build_selectors(C: int, R: int) -> np.ndarray:
    CR, CC = C * R, C * C
    m = np.arange(CC)
    i, j = m // C, m % C
    k = np.arange(CR)
    w = np.zeros((2, CR, CC), np.float32)
    w[0, :, :] = (k[:, None] // R == i[None, :])
    w[1, :, :] = (k[:, None] // R == j[None, :])
    return w


def _product_body(xt_ref, w_ref, o_ref):
    bB = xt_ref.shape[0]
    C, R, F = xt_ref.shape[1:]
    F2 = F // 2
    CR = C * R
    RF2 = R * F2
    dn = (((0,), (0,)), ((), ()))
    ik = lax.broadcasted_iota(jnp.int32, (CR, RF2), 0) % R
    im = lax.broadcasted_iota(jnp.int32, (CR, RF2), 1) % R
    msk = (ik == im).astype(jnp.float32)
    H = RF2 // 2
    col = lax.broadcasted_iota(jnp.int32, (CR, H), 1) // R
    idx_e = col * 2
    idx_o = idx_e + 1

    def deint(x64, idx):
        lo = jnp.take_along_axis(x64[:, : F // 2], idx, axis=1)
        hi = jnp.take_along_axis(x64[:, F // 2 :], idx, axis=1)
        return jnp.concatenate([lo, hi], axis=1)

    for b in range(bB):
        x64 = xt_ref[b].reshape(CR, F)
        xe4 = deint(x64, idx_e) * msk
        xo4 = deint(x64, idx_o) * msk
        p = lax.dot_general(xe4, w_ref[0], dn,
                            preferred_element_type=jnp.float32)
        q = lax.dot_general(xo4, w_ref[1], dn,
                            preferred_element_type=jnp.float32)
        o_ref[b] = (p + q).reshape(F2, R, C * C)


def kernel(log_likelihoods):
    B, F, C, R = log_likelihoods.shape
    F2 = F // 2
    CR, CC = C * R, C * C
    xt = jnp.transpose(log_likelihoods, (0, 2, 3, 1))
    w = jnp.asarray(_build_selectors(C, R))
    bB = 16
    yt = pl.pallas_call(
        _product_body,
        grid=(B // bB,),
        in_specs=[
            pl.BlockSpec((bB, C, R, F), lambda b: (b, 0, 0, 0)),
            pl.BlockSpec((2, CR, CC), lambda b: (0, 0, 0)),
        ],
        out_specs=pl.BlockSpec((bB, F2, R, CC), lambda b: (b, 0, 0, 0)),
        out_shape=jax.ShapeDtypeStruct((B, F2, R, CC), jnp.float32),
        compiler_params=pltpu.CompilerParams(
            dimension_semantics=("arbitrary",),
        ),
    )(xt, w)
    return jnp.transpose(yt, (0, 1, 3, 2))
